# Initial kernel scaffold; baseline (speedup 1.0000x reference)
import jax
import jax.numpy as jnp
from jax.experimental import pallas as pl


def kernel(band_id, table):
    raise NotImplementedError("write your pallas kernel here")


# speedup vs baseline: 2.9970x; 2.9970x over previous
import functools

import jax
import jax.numpy as jnp
from jax import lax
from jax.experimental import pallas as pl
from jax.experimental.pallas import tpu as pltpu
from jax.experimental.pallas import tpu_sc as plsc

_D = 32
_B = 16384 * 50

_info = plsc.get_sparse_core_info()
_NC, _NS = _info.num_cores, _info.num_subcores
_NW = _NC * _NS
_BPW = _B // _NW
_CH = 2560
_NCHUNK = _BPW // _CH

_mesh = plsc.VectorSubcoreMesh(core_axis_name="c", subcore_axis_name="s")


@functools.partial(
    pl.kernel,
    mesh=_mesh,
    compiler_params=pltpu.CompilerParams(use_tc_tiling_on_sc=False),
    out_type=jax.ShapeDtypeStruct((_B, _D), jnp.float32),
    scratch_types=[
        pltpu.VMEM((_CH,), jnp.int32),
        pltpu.VMEM((_CH, _D), jnp.float32),
        pltpu.SemaphoreType.DMA,
    ],
)
def _gather_rows(idx_hbm, table_hbm, out_hbm, idx_v, rows_v, sem):
    wid = lax.axis_index("s") * _NC + lax.axis_index("c")
    base = wid * _BPW

    def body(i, carry):
        off = base + i * _CH
        pltpu.sync_copy(idx_hbm.at[pl.ds(off, _CH)], idx_v)
        pltpu.async_copy(table_hbm.at[idx_v], rows_v, sem).wait()
        pltpu.sync_copy(rows_v, out_hbm.at[pl.ds(off, _CH)])
        return carry

    lax.fori_loop(0, _NCHUNK, body, 0)


def kernel(band_id, table):
    idx = band_id.reshape(-1).astype(jnp.int32)
    out = _gather_rows(idx, table)
    return out.reshape(band_id.shape + (_D,))


# speedup vs baseline: 2.9998x; 1.0010x over previous
import functools

import jax
import jax.numpy as jnp
from jax import lax
from jax.experimental import pallas as pl
from jax.experimental.pallas import tpu as pltpu
from jax.experimental.pallas import tpu_sc as plsc

_D = 32
_B = 16384 * 50

_info = plsc.get_sparse_core_info()
_NC, _NS = _info.num_cores, _info.num_subcores
_NW = _NC * _NS
_BPW = _B // _NW
_CH = 1600
_NCHUNK = _BPW // _CH

_mesh = plsc.VectorSubcoreMesh(core_axis_name="c", subcore_axis_name="s")


@functools.partial(
    pl.kernel,
    mesh=_mesh,
    compiler_params=pltpu.CompilerParams(use_tc_tiling_on_sc=False),
    out_type=jax.ShapeDtypeStruct((_B, _D), jnp.float32),
    scratch_types=[
        pltpu.VMEM((_CH,), jnp.int32),
        pltpu.VMEM((_CH,), jnp.int32),
        pltpu.VMEM((_CH, _D), jnp.float32),
        pltpu.VMEM((_CH, _D), jnp.float32),
        pltpu.SemaphoreType.DMA,
        pltpu.SemaphoreType.DMA,
    ],
)
def _gather_rows(idx_hbm, table_hbm, out_hbm, idx0, idx1, rows0, rows1,
                 gsem, osem):
    wid = lax.axis_index("s") * _NC + lax.axis_index("c")
    base = wid * _BPW
    idxs = (idx0, idx1)
    rows = (rows0, rows1)

    pltpu.sync_copy(idx_hbm.at[pl.ds(base, _CH)], idx0)
    g = pltpu.async_copy(table_hbm.at[idx0], rows0, gsem)
    pltpu.sync_copy(idx_hbm.at[pl.ds(base + _CH, _CH)], idx1)

    out_cp = [None] * _NCHUNK
    for i in range(_NCHUNK):
        g.wait()
        out_cp[i] = pltpu.async_copy(
            rows[i % 2], out_hbm.at[pl.ds(base + i * _CH, _CH)], osem)
        if i + 2 < _NCHUNK:
            pltpu.sync_copy(idx_hbm.at[pl.ds(base + (i + 2) * _CH, _CH)],
                            idxs[i % 2])
        if i + 1 < _NCHUNK:
            if i >= 1:
                out_cp[i - 1].wait()
            g = pltpu.async_copy(table_hbm.at[idxs[(i + 1) % 2]],
                                 rows[(i + 1) % 2], gsem)
    out_cp[_NCHUNK - 2].wait()
    out_cp[_NCHUNK - 1].wait()


def kernel(band_id, table):
    idx = band_id.reshape(-1).astype(jnp.int32)
    out = _gather_rows(idx, table)
    return out.reshape(band_id.shape + (_D,))


# speedup vs baseline: 6.1166x; 2.0390x over previous
import functools

import jax
import jax.numpy as jnp
from jax import lax
from jax.experimental import pallas as pl
from jax.experimental.pallas import tpu as pltpu
from jax.experimental.pallas import tpu_sc as plsc

_R = 16384
_S = 50
_D = 32

_info = plsc.get_sparse_core_info()
_NC, _NS = _info.num_cores, _info.num_subcores
_NW = _NC * _NS
_RPW = _R // _NW
_RCH = 32
_NCHUNK = _RPW // _RCH

_mesh = plsc.VectorSubcoreMesh(core_axis_name="c", subcore_axis_name="s")


@functools.partial(
    pl.kernel,
    mesh=_mesh,
    compiler_params=pltpu.CompilerParams(use_tc_tiling_on_sc=False),
    out_type=jax.ShapeDtypeStruct((_R, _S, _D), jnp.float32),
    scratch_types=[
        pltpu.VMEM((_RCH * _S,), jnp.int32),
        pltpu.VMEM((_RCH * _S, _D), jnp.float32),
        pltpu.SemaphoreType.DMA,
        pltpu.SemaphoreType.DMA,
    ],
)
def _gather_rows(idx_hbm, table_hbm, out_hbm, idx_v, rows_v, gsem, osem):
    wid = lax.axis_index("s") * _NC + lax.axis_index("c")
    base = wid * _RPW

    def body(i, carry):
        r0 = base + i * _RCH
        pltpu.sync_copy(idx_hbm.at[pl.ds(r0 * _S, _RCH * _S)], idx_v)
        pltpu.async_copy(table_hbm.at[idx_v], rows_v, gsem).wait()
        cps = [
            pltpu.async_copy(rows_v.at[pl.ds(j * _S, _S)],
                             out_hbm.at[r0 + j], osem)
            for j in range(_RCH)
        ]
        for cp in cps:
            cp.wait()
        return carry

    lax.fori_loop(0, _NCHUNK, body, 0)


def kernel(band_id, table):
    idx = band_id.reshape(-1).astype(jnp.int32)
    return _gather_rows(idx, table)
